# baseline (speedup 1.0000x reference)
import functools

import jax
import jax.numpy as jnp
from jax import lax
from jax.experimental import pallas as pl
from jax.experimental.pallas import tpu as pltpu
from jax.experimental.pallas import tpu_sc as plsc

_LANES = 16
_CHUNK = 256
_NC = 2
_NS = 16
_NW = _NC * _NS


_WIN = 4


def _sc_edge_pass_body(with_deg, n_pad, e_pad, *refs):
    if with_deg:
        (x_hbm, src_hbm, dst_hbm, zrows_hbm, zdeg_hbm,
         raw_hbm, deg0_hbm, deg1_hbm,
         acc_sh, deg_sh, idx_s, idx_d, rows_v, ones_v, *sems) = refs
    else:
        (x_hbm, src_hbm, dst_hbm, zrows_hbm,
         raw_hbm,
         acc_sh, idx_s, idx_d, rows_v, *sems) = refs
    si = sems[0:2]
    sg = sems[2:2 + _WIN]
    ss = sems[2 + _WIN:2 + 2 * _WIN]

    cid = lax.axis_index("c")
    sid = lax.axis_index("s")
    rpt = n_pad // _NS

    pltpu.sync_copy(zrows_hbm, acc_sh.at[pl.ds(sid * rpt, rpt)])
    if with_deg:
        pltpu.sync_copy(zdeg_hbm, deg_sh.at[pl.ds(sid * rpt, rpt)])
        for i in range(_CHUNK // 16):
            ones_v[pl.ds(i * 16, 16)] = jnp.ones((16,), jnp.float32)
    plsc.subcore_barrier()

    rows_per_sub = e_pad // (_NW * _CHUNK)
    nwin = rows_per_sub // _WIN
    base_row = (cid * _NS + sid) * rows_per_sub

    def fire_idx(h, w):
        r0 = base_row + w * _WIN
        pltpu.async_copy(src_hbm.at[pl.ds(r0, _WIN)], idx_s.at[h], si[h])
        pltpu.async_copy(dst_hbm.at[pl.ds(r0, _WIN)], idx_d.at[h], si[h])

    def wait_idx(h):
        r0 = base_row
        pltpu.make_async_copy(src_hbm.at[pl.ds(r0, _WIN)],
                              idx_s.at[h], si[h]).wait()
        pltpu.make_async_copy(dst_hbm.at[pl.ds(r0, _WIN)],
                              idx_d.at[h], si[h]).wait()

    def drain_bytes(j, sem):
        pltpu.make_async_copy(x_hbm.at[pl.ds(0, _CHUNK)],
                              rows_v.at[j], sem).wait()

    def drain_ones(j, sem):
        pltpu.make_async_copy(src_hbm.at[pl.ds(0, 1)],
                              idx_s.at[0, 0], sem).wait()

    def window(h, w):
        wait_idx(h)
        for j in range(_WIN):
            @pl.when(w >= 1)
            def _drain_ss():
                drain_bytes(j, ss[j])
                if with_deg:
                    drain_ones(j, ss[j])
            pltpu.async_copy(x_hbm.at[idx_s.at[h, j]], rows_v.at[j], sg[j])

        @pl.when(w + 1 < nwin)
        def _prefetch_idx():
            fire_idx(1 - h, w + 1)

        for j in range(_WIN):
            drain_bytes(j, sg[j])
            pltpu.async_copy(rows_v.at[j], acc_sh.at[idx_d.at[h, j]],
                             ss[j], add=True)
            if with_deg:
                pltpu.async_copy(ones_v, deg_sh.at[idx_s.at[h, j]],
                                 ss[j], add=True)

    fire_idx(0, 0)

    def body(g, carry):
        window(0, 2 * g)
        window(1, 2 * g + 1)
        return carry

    lax.fori_loop(0, nwin // 2, body, 0)
    for j in range(_WIN):
        drain_bytes(j, ss[j])
        if with_deg:
            drain_ones(j, ss[j])
    plsc.subcore_barrier()

    pltpu.sync_copy(acc_sh.at[pl.ds(sid * rpt, rpt)],
                    raw_hbm.at[cid, pl.ds(sid * rpt, rpt)])
    if with_deg:
        @pl.when(cid == 0)
        def _out0():
            pltpu.sync_copy(deg_sh.at[pl.ds(sid * rpt, rpt)],
                            deg0_hbm.at[pl.ds(sid * rpt, rpt)])

        @pl.when(cid == 1)
        def _out1():
            pltpu.sync_copy(deg_sh.at[pl.ds(sid * rpt, rpt)],
                            deg1_hbm.at[pl.ds(sid * rpt, rpt)])


def _sc_edge_pass(x_t, src2, dst2, n_pad, e_pad, with_deg=False):
    mesh = plsc.VectorSubcoreMesh(core_axis_name="c", subcore_axis_name="s")
    rpt = n_pad // _NS
    zrows = jnp.zeros((rpt, _LANES), jnp.float32)
    raw_t = jax.ShapeDtypeStruct((_NC, n_pad, _LANES), jnp.float32)
    idx_scr = [
        pltpu.VMEM((2, _WIN, _CHUNK), jnp.int32),
        pltpu.VMEM((2, _WIN, _CHUNK), jnp.int32),
        pltpu.VMEM((_WIN, _CHUNK, _LANES), jnp.float32),
    ]
    sems = [pltpu.SemaphoreType.DMA] * (2 + 2 * _WIN)
    if with_deg:
        out_type = (raw_t,
                    jax.ShapeDtypeStruct((n_pad,), jnp.float32),
                    jax.ShapeDtypeStruct((n_pad,), jnp.float32))
        scratch = ([pltpu.VMEM_SHARED((n_pad, _LANES), jnp.float32),
                    pltpu.VMEM_SHARED((n_pad,), jnp.float32)]
                   + idx_scr + [pltpu.VMEM((_CHUNK,), jnp.float32)] + sems)
        zdeg = jnp.zeros((rpt,), jnp.float32)
        args = (x_t, src2, dst2, zrows, zdeg)
    else:
        out_type = raw_t
        scratch = ([pltpu.VMEM_SHARED((n_pad, _LANES), jnp.float32)]
                   + idx_scr + sems)
        args = (x_t, src2, dst2, zrows)
    body = functools.partial(_sc_edge_pass_body, with_deg, n_pad, e_pad)
    return pl.kernel(
        body, out_type=out_type, mesh=mesh, scratch_types=scratch,
        compiler_params=pltpu.CompilerParams(use_tc_tiling_on_sc=False),
    )(*args)


_CR = 1568
_IOTA = None


def _row_idx(r):
    return jnp.full((16,), r, jnp.int32), lax.iota(jnp.int32, 16)


def _sc_prep_body(n_pad, ss_hbm, std_hbm, x0_hbm, ssv, stdv, x0v):
    wid = lax.axis_index("c") * _NS + lax.axis_index("s")
    rows = n_pad // _NW
    base0 = wid * rows

    def chunk(ch, carry):
        base = base0 + ch * _CR
        pltpu.sync_copy(ss_hbm.at[pl.ds(base, _CR)], ssv)
        pltpu.sync_copy(std_hbm.at[pl.ds(base, _CR)], stdv)

        def row(r, c2):
            splat, io = _row_idx(r)
            f = plsc.load_gather(stdv, [splat])
            x = plsc.load_gather(ssv, [splat, io])
            plsc.store_scatter(x0v, [splat, io], f * x)
            return c2

        lax.fori_loop(0, _CR, row, 0, unroll=4)
        pltpu.sync_copy(x0v, x0_hbm.at[pl.ds(base, _CR)])
        return carry

    lax.fori_loop(0, rows // _CR, chunk, 0)


def _sc_prep(ss2, std, n_pad):
    mesh = plsc.VectorSubcoreMesh(core_axis_name="c", subcore_axis_name="s")
    out_type = jax.ShapeDtypeStruct((n_pad, _LANES), jnp.float32)
    scratch = [
        pltpu.VMEM((_CR, _LANES), jnp.float32),
        pltpu.VMEM((_CR,), jnp.float32),
        pltpu.VMEM((_CR, _LANES), jnp.float32),
    ]
    body = functools.partial(_sc_prep_body, n_pad)
    return pl.kernel(
        body, out_type=out_type, mesh=mesh, scratch_types=scratch,
        compiler_params=pltpu.CompilerParams(use_tc_tiling_on_sc=False,
                                            needs_layout_passes=False),
    )(ss2, std)


def _sc_combine_body(n_pad, x_hbm, raw_hbm, a_hbm, b_hbm, xn_hbm,
                     xv, r0v, r1v, av, bv, xnv):
    wid = lax.axis_index("c") * _NS + lax.axis_index("s")
    rows = n_pad // _NW
    base0 = wid * rows

    def chunk(ch, carry):
        base = base0 + ch * _CR
        pltpu.sync_copy(x_hbm.at[pl.ds(base, _CR)], xv)
        pltpu.sync_copy(raw_hbm.at[0, pl.ds(base, _CR)], r0v)
        pltpu.sync_copy(raw_hbm.at[1, pl.ds(base, _CR)], r1v)
        pltpu.sync_copy(a_hbm.at[pl.ds(base, _CR)], av)
        pltpu.sync_copy(b_hbm.at[pl.ds(base, _CR)], bv)

        def row(r, c2):
            splat, io = _row_idx(r)
            fa = plsc.load_gather(av, [splat])
            fb = plsc.load_gather(bv, [splat])
            x = plsc.load_gather(xv, [splat, io])
            r0 = plsc.load_gather(r0v, [splat, io])
            r1 = plsc.load_gather(r1v, [splat, io])
            plsc.store_scatter(xnv, [splat, io], fa * x + fb * (r0 + r1))
            return c2

        lax.fori_loop(0, _CR, row, 0, unroll=4)
        pltpu.sync_copy(xnv, xn_hbm.at[pl.ds(base, _CR)])
        return carry

    lax.fori_loop(0, rows // _CR, chunk, 0)


def _sc_combine(x, raw, a, b, n_pad):
    mesh = plsc.VectorSubcoreMesh(core_axis_name="c", subcore_axis_name="s")
    out_type = jax.ShapeDtypeStruct((n_pad, _LANES), jnp.float32)
    scratch = (
        [pltpu.VMEM((_CR, _LANES), jnp.float32)] * 3
        + [pltpu.VMEM((_CR,), jnp.float32)] * 2
        + [pltpu.VMEM((_CR, _LANES), jnp.float32)]
    )
    body = functools.partial(_sc_combine_body, n_pad)
    return pl.kernel(
        body, out_type=out_type, mesh=mesh, scratch_types=scratch,
        compiler_params=pltpu.CompilerParams(use_tc_tiling_on_sc=False,
                                            needs_layout_passes=False),
    )(x, raw, a, b)


def _sc_final_body(n_pad, x_hbm, raw_hbm, a_hbm, b_hbm, p_hbm, m_hbm, o_hbm,
                   xv, r0v, r1v, av, bv, pv, mv, ov):
    wid = lax.axis_index("c") * _NS + lax.axis_index("s")
    rows = n_pad // _NW
    base0 = wid * rows

    def chunk(ch, carry):
        base = base0 + ch * _CR
        pltpu.sync_copy(x_hbm.at[pl.ds(base, _CR)], xv)
        pltpu.sync_copy(raw_hbm.at[0, pl.ds(base, _CR)], r0v)
        pltpu.sync_copy(raw_hbm.at[1, pl.ds(base, _CR)], r1v)
        pltpu.sync_copy(a_hbm.at[pl.ds(base, _CR)], av)
        pltpu.sync_copy(b_hbm.at[pl.ds(base, _CR)], bv)
        pltpu.sync_copy(p_hbm.at[pl.ds(base, _CR)], pv)
        pltpu.sync_copy(m_hbm.at[pl.ds(base, _CR)], mv)

        def row(r, c2):
            splat, io = _row_idx(r)
            fa = plsc.load_gather(av, [splat])
            fb = plsc.load_gather(bv, [splat])
            fp = plsc.load_gather(pv, [splat])
            fm = plsc.load_gather(mv, [splat])
            x = plsc.load_gather(xv, [splat, io])
            r0 = plsc.load_gather(r0v, [splat, io])
            r1 = plsc.load_gather(r1v, [splat, io])
            x2 = fa * x + fb * (r0 + r1)
            plsc.store_scatter(ov, [io, splat], fp * x2 + fm)
            return c2

        lax.fori_loop(0, _CR, row, 0, unroll=4)
        pltpu.sync_copy(ov, o_hbm.at[:, pl.ds(base, _CR)])
        return carry

    lax.fori_loop(0, rows // _CR, chunk, 0)


def _sc_final(x, raw, a, b, p, m, n_pad):
    mesh = plsc.VectorSubcoreMesh(core_axis_name="c", subcore_axis_name="s")
    out_type = jax.ShapeDtypeStruct((_LANES, n_pad), jnp.float32)
    scratch = (
        [pltpu.VMEM((_CR, _LANES), jnp.float32)] * 3
        + [pltpu.VMEM((_CR,), jnp.float32)] * 4
        + [pltpu.VMEM((_LANES, _CR), jnp.float32)]
    )
    body = functools.partial(_sc_final_body, n_pad)
    return pl.kernel(
        body, out_type=out_type, mesh=mesh, scratch_types=scratch,
        compiler_params=pltpu.CompilerParams(use_tc_tiling_on_sc=False,
                                            needs_layout_passes=False),
    )(x, raw, a, b, p, m)


def _factor_body(params_ref, deg0_ref, deg1_ref, diag_ref, pdiag_ref,
                 std_ref, a1_ref, b1_ref, a2_ref, b2_ref, pdq_ref):
    s1, n1, g1 = params_ref[0], params_ref[1], params_ref[2]
    s2, n2, g2 = params_ref[3], params_ref[4], params_ref[5]
    deg = jnp.maximum(deg0_ref[...] + deg1_ref[...], 1.0)
    ld = jnp.log(deg)
    a1_ref[...] = s1 * jnp.exp(g1 * ld)
    b1_ref[...] = n1 * jnp.exp((g1 - 1.0) * ld)
    a2_ref[...] = s2 * jnp.exp(g2 * ld)
    b2_ref[...] = n2 * jnp.exp((g2 - 1.0) * ld)
    std_ref[...] = jax.nn.softplus(diag_ref[...])
    pdq_ref[...] = jax.nn.softplus(pdiag_ref[...])


def _std_body(diag_ref, std_ref):
    std_ref[...] = jax.nn.softplus(diag_ref[...])


def _tc_std(diag_pad, n_pad):
    m = n_pad // 128
    full = pl.BlockSpec((m, 128), lambda: (0, 0))
    out = pl.pallas_call(
        _std_body, in_specs=[full], out_specs=full,
        out_shape=jax.ShapeDtypeStruct((m, 128), jnp.float32),
    )(diag_pad.reshape(m, 128))
    return out.reshape(n_pad)


def _tc_factors(params, deg0, deg1, diag_pad, pdiag_pad, n_pad):
    m = n_pad // 128
    shp = jax.ShapeDtypeStruct((m, 128), jnp.float32)
    full = pl.BlockSpec((m, 128), lambda: (0, 0))
    outs = pl.pallas_call(
        _factor_body,
        in_specs=[pl.BlockSpec(memory_space=pltpu.SMEM)] + [full] * 4,
        out_specs=[full] * 6,
        out_shape=[shp] * 6,
    )(params, deg0.reshape(m, 128), deg1.reshape(m, 128),
      diag_pad.reshape(m, 128), pdiag_pad.reshape(m, 128))
    return [o.reshape(n_pad) for o in outs]


def kernel(standard_sample, mean_param, diag_param, post_diag_param,
           alpha1, alpha2, gamma_param, edge_index):
    S, N = standard_sample.shape
    E = edge_index.shape[1]
    bn = 2048
    n_pad = ((N + 1 + bn - 1) // bn) * bn
    e_align = _NW * _CHUNK * _WIN * 2
    e_pad = ((E + e_align - 1) // e_align) * e_align

    ss_t = jnp.pad(standard_sample.T, ((0, n_pad - N), (0, _LANES - S)))
    diag_pad = jnp.pad(diag_param, (0, n_pad - N))
    pdiag_pad = jnp.pad(post_diag_param, (0, n_pad - N))
    mean_pad = jnp.pad(mean_param, (0, n_pad - N))
    src2 = jnp.pad(edge_index[0], (0, e_pad - E),
                   constant_values=N).reshape(-1, _CHUNK)
    dst2 = jnp.pad(edge_index[1], (0, e_pad - E),
                   constant_values=N).reshape(-1, _CHUNK)
    sw = jnp.exp(alpha1)
    nw = sw * jnp.tanh(alpha2)
    g = jax.nn.sigmoid(gamma_param)
    params = jnp.stack([sw[0], nw[0], g[0], sw[1], nw[1], g[1]])

    stdf = _tc_std(diag_pad, n_pad)
    x0 = _sc_prep(ss_t, stdf, n_pad)
    raw1, deg0, deg1 = _sc_edge_pass(x0, src2, dst2, n_pad, e_pad,
                                     with_deg=True)
    _, a1, b1, a2, b2, pdq = _tc_factors(
        params, deg0, deg1, diag_pad, pdiag_pad, n_pad)
    x1 = _sc_combine(x0, raw1, a1, b1, n_pad)
    raw2 = _sc_edge_pass(x1, src2, dst2, n_pad, e_pad)
    out_t = _sc_final(x1, raw2, a2, b2, pdq, mean_pad, n_pad)
    return out_t[:S, :N]


# speedup vs baseline: 1.0307x; 1.0307x over previous
import functools

import jax
import jax.numpy as jnp
from jax import lax
from jax.experimental import pallas as pl
from jax.experimental.pallas import tpu as pltpu
from jax.experimental.pallas import tpu_sc as plsc

_LANES = 16
_CHUNK = 256
_NC = 2
_NS = 16
_NW = _NC * _NS


_WIN = 4


def _sc_edge_pass_body(with_deg, n_pad, e_pad, *refs):
    if with_deg:
        (x_hbm, src_hbm, dst_hbm, zrows_hbm, zdeg_hbm,
         raw_hbm, deg0_hbm, deg1_hbm,
         acc_sh, deg_sh, idx_s, idx_d, rows_v, ones_v, *sems) = refs
    else:
        (x_hbm, src_hbm, dst_hbm, zrows_hbm,
         raw_hbm,
         acc_sh, idx_s, idx_d, rows_v, *sems) = refs
    si = sems[0:2]
    sg = sems[2:2 + _WIN]
    ss = sems[2 + _WIN:2 + 2 * _WIN]

    cid = lax.axis_index("c")
    sid = lax.axis_index("s")
    rpt = n_pad // _NS

    pltpu.sync_copy(zrows_hbm, acc_sh.at[pl.ds(sid * rpt, rpt)])
    if with_deg:
        pltpu.sync_copy(zdeg_hbm, deg_sh.at[pl.ds(sid * rpt, rpt)])
        for i in range(_CHUNK // 16):
            ones_v[pl.ds(i * 16, 16)] = jnp.ones((16,), jnp.float32)
    plsc.subcore_barrier()

    rows_per_sub = e_pad // (_NW * _CHUNK)
    nwin = rows_per_sub // _WIN
    base_row = (cid * _NS + sid) * rows_per_sub

    def fire_idx(h, w):
        r0 = base_row + w * _WIN
        pltpu.async_copy(src_hbm.at[pl.ds(r0, _WIN)], idx_s.at[h], si[h])
        pltpu.async_copy(dst_hbm.at[pl.ds(r0, _WIN)], idx_d.at[h], si[h])

    def wait_idx(h):
        r0 = base_row
        pltpu.make_async_copy(src_hbm.at[pl.ds(r0, _WIN)],
                              idx_s.at[h], si[h]).wait()
        pltpu.make_async_copy(dst_hbm.at[pl.ds(r0, _WIN)],
                              idx_d.at[h], si[h]).wait()

    def drain_bytes(j, sem):
        pltpu.make_async_copy(x_hbm.at[pl.ds(0, _CHUNK)],
                              rows_v.at[j], sem).wait()

    def drain_ones(j, sem):
        pltpu.make_async_copy(src_hbm.at[pl.ds(0, 1)],
                              idx_s.at[0, 0], sem).wait()

    def window(h, w):
        wait_idx(h)
        for j in range(_WIN):
            @pl.when(w >= 1)
            def _drain_ss():
                drain_bytes(j, ss[j])
                if with_deg:
                    drain_ones(j, ss[j])
            pltpu.async_copy(x_hbm.at[idx_s.at[h, j]], rows_v.at[j], sg[j])

        @pl.when(w + 1 < nwin)
        def _prefetch_idx():
            fire_idx(1 - h, w + 1)

        for j in range(_WIN):
            drain_bytes(j, sg[j])
            pltpu.async_copy(rows_v.at[j], acc_sh.at[idx_d.at[h, j]],
                             ss[j], add=True)
            if with_deg:
                pltpu.async_copy(ones_v, deg_sh.at[idx_s.at[h, j]],
                                 ss[j], add=True)

    fire_idx(0, 0)

    def body(g, carry):
        window(0, 2 * g)
        window(1, 2 * g + 1)
        return carry

    lax.fori_loop(0, nwin // 2, body, 0)
    for j in range(_WIN):
        drain_bytes(j, ss[j])
        if with_deg:
            drain_ones(j, ss[j])
    plsc.subcore_barrier()

    pltpu.sync_copy(acc_sh.at[pl.ds(sid * rpt, rpt)],
                    raw_hbm.at[cid, pl.ds(sid * rpt, rpt)])
    if with_deg:
        @pl.when(cid == 0)
        def _out0():
            pltpu.sync_copy(deg_sh.at[pl.ds(sid * rpt, rpt)],
                            deg0_hbm.at[pl.ds(sid * rpt, rpt)])

        @pl.when(cid == 1)
        def _out1():
            pltpu.sync_copy(deg_sh.at[pl.ds(sid * rpt, rpt)],
                            deg1_hbm.at[pl.ds(sid * rpt, rpt)])


def _sc_edge_pass(x_t, src2, dst2, n_pad, e_pad, with_deg=False):
    mesh = plsc.VectorSubcoreMesh(core_axis_name="c", subcore_axis_name="s")
    rpt = n_pad // _NS
    zrows = jnp.zeros((rpt, _LANES), jnp.float32)
    raw_t = jax.ShapeDtypeStruct((_NC, n_pad, _LANES), jnp.float32)
    idx_scr = [
        pltpu.VMEM((2, _WIN, _CHUNK), jnp.int32),
        pltpu.VMEM((2, _WIN, _CHUNK), jnp.int32),
        pltpu.VMEM((_WIN, _CHUNK, _LANES), jnp.float32),
    ]
    sems = [pltpu.SemaphoreType.DMA] * (2 + 2 * _WIN)
    if with_deg:
        out_type = (raw_t,
                    jax.ShapeDtypeStruct((n_pad,), jnp.float32),
                    jax.ShapeDtypeStruct((n_pad,), jnp.float32))
        scratch = ([pltpu.VMEM_SHARED((n_pad, _LANES), jnp.float32),
                    pltpu.VMEM_SHARED((n_pad,), jnp.float32)]
                   + idx_scr + [pltpu.VMEM((_CHUNK,), jnp.float32)] + sems)
        zdeg = jnp.zeros((rpt,), jnp.float32)
        args = (x_t, src2, dst2, zrows, zdeg)
    else:
        out_type = raw_t
        scratch = ([pltpu.VMEM_SHARED((n_pad, _LANES), jnp.float32)]
                   + idx_scr + sems)
        args = (x_t, src2, dst2, zrows)
    body = functools.partial(_sc_edge_pass_body, with_deg, n_pad, e_pad)
    return pl.kernel(
        body, out_type=out_type, mesh=mesh, scratch_types=scratch,
        compiler_params=pltpu.CompilerParams(use_tc_tiling_on_sc=False),
    )(*args)


_CR = 784
_IOTA = None


def _row_idx(r):
    return jnp.full((16,), r, jnp.int32), lax.iota(jnp.int32, 16)


def _sc_prep_body(n_pad, ss_hbm, std_hbm, x0_hbm, ssv, stdv, x0v):
    wid = lax.axis_index("c") * _NS + lax.axis_index("s")
    rows = n_pad // _NW
    base0 = wid * rows

    def chunk(ch, carry):
        base = base0 + ch * _CR
        pltpu.sync_copy(ss_hbm.at[pl.ds(base, _CR)], ssv)
        pltpu.sync_copy(std_hbm.at[pl.ds(base, _CR)], stdv)

        def row(r, c2):
            splat, io = _row_idx(r)
            f = plsc.load_gather(stdv, [splat])
            x = plsc.load_gather(ssv, [splat, io])
            plsc.store_scatter(x0v, [splat, io], f * x)
            return c2

        lax.fori_loop(0, _CR, row, 0, unroll=4)
        pltpu.sync_copy(x0v, x0_hbm.at[pl.ds(base, _CR)])
        return carry

    lax.fori_loop(0, rows // _CR, chunk, 0)


def _sc_prep(ss2, std, n_pad):
    mesh = plsc.VectorSubcoreMesh(core_axis_name="c", subcore_axis_name="s")
    out_type = jax.ShapeDtypeStruct((n_pad, _LANES), jnp.float32)
    scratch = [
        pltpu.VMEM((_CR, _LANES), jnp.float32),
        pltpu.VMEM((_CR,), jnp.float32),
        pltpu.VMEM((_CR, _LANES), jnp.float32),
    ]
    body = functools.partial(_sc_prep_body, n_pad)
    return pl.kernel(
        body, out_type=out_type, mesh=mesh, scratch_types=scratch,
        compiler_params=pltpu.CompilerParams(use_tc_tiling_on_sc=False,
                                            needs_layout_passes=False),
    )(ss2, std)


def _sc_combine_body(n_pad, x_hbm, raw_hbm, a_hbm, b_hbm, xn_hbm,
                     xv, r0v, r1v, av, bv, xnv, sin0, sin1, sout):
    wid = lax.axis_index("c") * _NS + lax.axis_index("s")
    rows = n_pad // _NW
    base0 = wid * rows
    nch = rows // _CR
    sin = (sin0, sin1)

    pltpu.sync_copy(a_hbm.at[pl.ds(base0, rows)], av)
    pltpu.sync_copy(b_hbm.at[pl.ds(base0, rows)], bv)

    def fire_in(b, ch):
        base = base0 + ch * _CR
        pltpu.async_copy(x_hbm.at[pl.ds(base, _CR)], xv.at[b], sin[b])
        pltpu.async_copy(raw_hbm.at[0, pl.ds(base, _CR)], r0v.at[b], sin[b])
        pltpu.async_copy(raw_hbm.at[1, pl.ds(base, _CR)], r1v.at[b], sin[b])

    def wait_in(b):
        for dst in (xv.at[b], r0v.at[b], r1v.at[b]):
            pltpu.make_async_copy(x_hbm.at[pl.ds(0, _CR)], dst, sin[b]).wait()

    def drain_out(b):
        pltpu.make_async_copy(x_hbm.at[pl.ds(0, _CR)], xnv.at[b],
                              sout).wait()

    fire_in(0, 0)
    for ch in range(nch):
        b = ch % 2
        if ch + 1 < nch:
            fire_in(1 - b, ch + 1)
        wait_in(b)
        if ch >= 2:
            drain_out(b)
        abase = ch * _CR

        def row(r, c2):
            splat, io = _row_idx(r)
            asp = jnp.full((16,), abase, jnp.int32) + splat
            bsp = jnp.full((16,), b, jnp.int32)
            fa = plsc.load_gather(av, [asp])
            fb = plsc.load_gather(bv, [asp])
            x = plsc.load_gather(xv, [bsp, splat, io])
            r0 = plsc.load_gather(r0v, [bsp, splat, io])
            r1 = plsc.load_gather(r1v, [bsp, splat, io])
            plsc.store_scatter(xnv, [bsp, splat, io],
                               fa * x + fb * (r0 + r1))
            return c2

        lax.fori_loop(0, _CR, row, 0, unroll=4)
        pltpu.async_copy(xnv.at[b], xn_hbm.at[pl.ds(base0 + ch * _CR, _CR)],
                         sout)
    for _ in range(min(nch, 2)):
        drain_out(0)


def _sc_combine(x, raw, a, b, n_pad):
    mesh = plsc.VectorSubcoreMesh(core_axis_name="c", subcore_axis_name="s")
    rows = n_pad // _NW
    out_type = jax.ShapeDtypeStruct((n_pad, _LANES), jnp.float32)
    scratch = (
        [pltpu.VMEM((2, _CR, _LANES), jnp.float32)] * 3
        + [pltpu.VMEM((rows,), jnp.float32)] * 2
        + [pltpu.VMEM((2, _CR, _LANES), jnp.float32)]
        + [pltpu.SemaphoreType.DMA] * 3
    )
    body = functools.partial(_sc_combine_body, n_pad)
    return pl.kernel(
        body, out_type=out_type, mesh=mesh, scratch_types=scratch,
        compiler_params=pltpu.CompilerParams(use_tc_tiling_on_sc=False,
                                            needs_layout_passes=False),
    )(x, raw, a, b)


def _sc_final_body(n_pad, x_hbm, raw_hbm, a_hbm, b_hbm, p_hbm, m_hbm, o_hbm,
                   xv, r0v, r1v, av, bv, pv, mv, ov):
    wid = lax.axis_index("c") * _NS + lax.axis_index("s")
    rows = n_pad // _NW
    base0 = wid * rows

    def chunk(ch, carry):
        base = base0 + ch * _CR
        pltpu.sync_copy(x_hbm.at[pl.ds(base, _CR)], xv)
        pltpu.sync_copy(raw_hbm.at[0, pl.ds(base, _CR)], r0v)
        pltpu.sync_copy(raw_hbm.at[1, pl.ds(base, _CR)], r1v)
        pltpu.sync_copy(a_hbm.at[pl.ds(base, _CR)], av)
        pltpu.sync_copy(b_hbm.at[pl.ds(base, _CR)], bv)
        pltpu.sync_copy(p_hbm.at[pl.ds(base, _CR)], pv)
        pltpu.sync_copy(m_hbm.at[pl.ds(base, _CR)], mv)

        def row(r, c2):
            splat, io = _row_idx(r)
            fa = plsc.load_gather(av, [splat])
            fb = plsc.load_gather(bv, [splat])
            fp = plsc.load_gather(pv, [splat])
            fm = plsc.load_gather(mv, [splat])
            x = plsc.load_gather(xv, [splat, io])
            r0 = plsc.load_gather(r0v, [splat, io])
            r1 = plsc.load_gather(r1v, [splat, io])
            x2 = fa * x + fb * (r0 + r1)
            plsc.store_scatter(ov, [io, splat], fp * x2 + fm)
            return c2

        lax.fori_loop(0, _CR, row, 0, unroll=4)
        pltpu.sync_copy(ov, o_hbm.at[:, pl.ds(base, _CR)])
        return carry

    lax.fori_loop(0, rows // _CR, chunk, 0)


def _sc_final(x, raw, a, b, p, m, n_pad):
    mesh = plsc.VectorSubcoreMesh(core_axis_name="c", subcore_axis_name="s")
    out_type = jax.ShapeDtypeStruct((_LANES, n_pad), jnp.float32)
    scratch = (
        [pltpu.VMEM((_CR, _LANES), jnp.float32)] * 3
        + [pltpu.VMEM((_CR,), jnp.float32)] * 4
        + [pltpu.VMEM((_LANES, _CR), jnp.float32)]
    )
    body = functools.partial(_sc_final_body, n_pad)
    return pl.kernel(
        body, out_type=out_type, mesh=mesh, scratch_types=scratch,
        compiler_params=pltpu.CompilerParams(use_tc_tiling_on_sc=False,
                                            needs_layout_passes=False),
    )(x, raw, a, b, p, m)


def _factor_body(params_ref, deg0_ref, deg1_ref, diag_ref, pdiag_ref,
                 std_ref, a1_ref, b1_ref, a2_ref, b2_ref, pdq_ref):
    s1, n1, g1 = params_ref[0], params_ref[1], params_ref[2]
    s2, n2, g2 = params_ref[3], params_ref[4], params_ref[5]
    deg = jnp.maximum(deg0_ref[...] + deg1_ref[...], 1.0)
    ld = jnp.log(deg)
    a1_ref[...] = s1 * jnp.exp(g1 * ld)
    b1_ref[...] = n1 * jnp.exp((g1 - 1.0) * ld)
    a2_ref[...] = s2 * jnp.exp(g2 * ld)
    b2_ref[...] = n2 * jnp.exp((g2 - 1.0) * ld)
    std_ref[...] = jax.nn.softplus(diag_ref[...])
    pdq_ref[...] = jax.nn.softplus(pdiag_ref[...])


def _std_body(diag_ref, std_ref):
    std_ref[...] = jax.nn.softplus(diag_ref[...])


def _tc_std(diag_pad, n_pad):
    m = n_pad // 128
    full = pl.BlockSpec((m, 128), lambda: (0, 0))
    out = pl.pallas_call(
        _std_body, in_specs=[full], out_specs=full,
        out_shape=jax.ShapeDtypeStruct((m, 128), jnp.float32),
    )(diag_pad.reshape(m, 128))
    return out.reshape(n_pad)


def _tc_factors(params, deg0, deg1, diag_pad, pdiag_pad, n_pad):
    m = n_pad // 128
    shp = jax.ShapeDtypeStruct((m, 128), jnp.float32)
    full = pl.BlockSpec((m, 128), lambda: (0, 0))
    outs = pl.pallas_call(
        _factor_body,
        in_specs=[pl.BlockSpec(memory_space=pltpu.SMEM)] + [full] * 4,
        out_specs=[full] * 6,
        out_shape=[shp] * 6,
    )(params, deg0.reshape(m, 128), deg1.reshape(m, 128),
      diag_pad.reshape(m, 128), pdiag_pad.reshape(m, 128))
    return [o.reshape(n_pad) for o in outs]


def kernel(standard_sample, mean_param, diag_param, post_diag_param,
           alpha1, alpha2, gamma_param, edge_index):
    S, N = standard_sample.shape
    E = edge_index.shape[1]
    bn = 2048
    n_pad = ((N + 1 + bn - 1) // bn) * bn
    e_align = _NW * _CHUNK * _WIN * 2
    e_pad = ((E + e_align - 1) // e_align) * e_align

    ss_t = jnp.pad(standard_sample.T, ((0, n_pad - N), (0, _LANES - S)))
    diag_pad = jnp.pad(diag_param, (0, n_pad - N))
    pdiag_pad = jnp.pad(post_diag_param, (0, n_pad - N))
    mean_pad = jnp.pad(mean_param, (0, n_pad - N))
    src2 = jnp.pad(edge_index[0], (0, e_pad - E),
                   constant_values=N).reshape(-1, _CHUNK)
    dst2 = jnp.pad(edge_index[1], (0, e_pad - E),
                   constant_values=N).reshape(-1, _CHUNK)
    sw = jnp.exp(alpha1)
    nw = sw * jnp.tanh(alpha2)
    g = jax.nn.sigmoid(gamma_param)
    params = jnp.stack([sw[0], nw[0], g[0], sw[1], nw[1], g[1]])

    stdf = _tc_std(diag_pad, n_pad)
    x0 = _sc_prep(ss_t, stdf, n_pad)
    raw1, deg0, deg1 = _sc_edge_pass(x0, src2, dst2, n_pad, e_pad,
                                     with_deg=True)
    _, a1, b1, a2, b2, pdq = _tc_factors(
        params, deg0, deg1, diag_pad, pdiag_pad, n_pad)
    x1 = _sc_combine(x0, raw1, a1, b1, n_pad)
    raw2 = _sc_edge_pass(x1, src2, dst2, n_pad, e_pad)
    out_t = _sc_final(x1, raw2, a2, b2, pdq, mean_pad, n_pad)
    return out_t[:S, :N]


# speedup vs baseline: 1.0535x; 1.0221x over previous
import functools

import jax
import jax.numpy as jnp
from jax import lax
from jax.experimental import pallas as pl
from jax.experimental.pallas import tpu as pltpu
from jax.experimental.pallas import tpu_sc as plsc

_LANES = 16
_CHUNK = 256
_NC = 2
_NS = 16
_NW = _NC * _NS


_WIN = 4


def _sc_edge_pass_body(with_deg, n_pad, e_pad, *refs):
    if with_deg:
        (x_hbm, src_hbm, dst_hbm, zrows_hbm, zdeg_hbm,
         raw_hbm, deg0_hbm, deg1_hbm,
         acc_sh, deg_sh, idx_s, idx_d, rows_v, ones_v, *sems) = refs
    else:
        (x_hbm, src_hbm, dst_hbm, zrows_hbm,
         raw_hbm,
         acc_sh, idx_s, idx_d, rows_v, *sems) = refs
    si = sems[0:2]
    sg = sems[2:2 + _WIN]
    ss = sems[2 + _WIN:2 + 2 * _WIN]

    cid = lax.axis_index("c")
    sid = lax.axis_index("s")
    rpt = n_pad // _NS

    pltpu.sync_copy(zrows_hbm, acc_sh.at[pl.ds(sid * rpt, rpt)])
    if with_deg:
        pltpu.sync_copy(zdeg_hbm, deg_sh.at[pl.ds(sid * rpt, rpt)])
        for i in range(_CHUNK // 16):
            ones_v[pl.ds(i * 16, 16)] = jnp.ones((16,), jnp.float32)
    plsc.subcore_barrier()

    rows_per_sub = e_pad // (_NW * _CHUNK)
    nwin = rows_per_sub // _WIN
    base_row = (cid * _NS + sid) * rows_per_sub

    def fire_idx(h, w):
        r0 = base_row + w * _WIN
        pltpu.async_copy(src_hbm.at[pl.ds(r0, _WIN)], idx_s.at[h], si[h])
        pltpu.async_copy(dst_hbm.at[pl.ds(r0, _WIN)], idx_d.at[h], si[h])

    def wait_idx(h):
        r0 = base_row
        pltpu.make_async_copy(src_hbm.at[pl.ds(r0, _WIN)],
                              idx_s.at[h], si[h]).wait()
        pltpu.make_async_copy(dst_hbm.at[pl.ds(r0, _WIN)],
                              idx_d.at[h], si[h]).wait()

    def drain_bytes(j, sem):
        pltpu.make_async_copy(x_hbm.at[pl.ds(0, _CHUNK)],
                              rows_v.at[j], sem).wait()

    def drain_ones(j, sem):
        pltpu.make_async_copy(src_hbm.at[pl.ds(0, 1)],
                              idx_s.at[0, 0], sem).wait()

    def window(h, w):
        wait_idx(h)
        for j in range(_WIN):
            @pl.when(w >= 1)
            def _drain_ss():
                drain_bytes(j, ss[j])
                if with_deg:
                    drain_ones(j, ss[j])
            pltpu.async_copy(x_hbm.at[idx_s.at[h, j]], rows_v.at[j], sg[j])

        @pl.when(w + 1 < nwin)
        def _prefetch_idx():
            fire_idx(1 - h, w + 1)

        for j in range(_WIN):
            drain_bytes(j, sg[j])
            pltpu.async_copy(rows_v.at[j], acc_sh.at[idx_d.at[h, j]],
                             ss[j], add=True)
            if with_deg:
                pltpu.async_copy(ones_v, deg_sh.at[idx_s.at[h, j]],
                                 ss[j], add=True)

    fire_idx(0, 0)

    def body(g, carry):
        window(0, 2 * g)
        window(1, 2 * g + 1)
        return carry

    lax.fori_loop(0, nwin // 2, body, 0)
    for j in range(_WIN):
        drain_bytes(j, ss[j])
        if with_deg:
            drain_ones(j, ss[j])
    plsc.subcore_barrier()

    pltpu.sync_copy(acc_sh.at[pl.ds(sid * rpt, rpt)],
                    raw_hbm.at[cid, pl.ds(sid * rpt, rpt)])
    if with_deg:
        @pl.when(cid == 0)
        def _out0():
            pltpu.sync_copy(deg_sh.at[pl.ds(sid * rpt, rpt)],
                            deg0_hbm.at[pl.ds(sid * rpt, rpt)])

        @pl.when(cid == 1)
        def _out1():
            pltpu.sync_copy(deg_sh.at[pl.ds(sid * rpt, rpt)],
                            deg1_hbm.at[pl.ds(sid * rpt, rpt)])


def _sc_edge_pass(x_t, src2, dst2, n_pad, e_pad, with_deg=False):
    mesh = plsc.VectorSubcoreMesh(core_axis_name="c", subcore_axis_name="s")
    rpt = n_pad // _NS
    zrows = jnp.zeros((rpt, _LANES), jnp.float32)
    raw_t = jax.ShapeDtypeStruct((_NC, n_pad, _LANES), jnp.float32)
    idx_scr = [
        pltpu.VMEM((2, _WIN, _CHUNK), jnp.int32),
        pltpu.VMEM((2, _WIN, _CHUNK), jnp.int32),
        pltpu.VMEM((_WIN, _CHUNK, _LANES), jnp.float32),
    ]
    sems = [pltpu.SemaphoreType.DMA] * (2 + 2 * _WIN)
    if with_deg:
        out_type = (raw_t,
                    jax.ShapeDtypeStruct((n_pad,), jnp.float32),
                    jax.ShapeDtypeStruct((n_pad,), jnp.float32))
        scratch = ([pltpu.VMEM_SHARED((n_pad, _LANES), jnp.float32),
                    pltpu.VMEM_SHARED((n_pad,), jnp.float32)]
                   + idx_scr + [pltpu.VMEM((_CHUNK,), jnp.float32)] + sems)
        zdeg = jnp.zeros((rpt,), jnp.float32)
        args = (x_t, src2, dst2, zrows, zdeg)
    else:
        out_type = raw_t
        scratch = ([pltpu.VMEM_SHARED((n_pad, _LANES), jnp.float32)]
                   + idx_scr + sems)
        args = (x_t, src2, dst2, zrows)
    body = functools.partial(_sc_edge_pass_body, with_deg, n_pad, e_pad)
    return pl.kernel(
        body, out_type=out_type, mesh=mesh, scratch_types=scratch,
        compiler_params=pltpu.CompilerParams(use_tc_tiling_on_sc=False),
    )(*args)


_CR = 784
_IOTA = None


def _row_idx(r):
    return jnp.full((16,), r, jnp.int32), lax.iota(jnp.int32, 16)


def _sc_prep_body(n_pad, ss_hbm, std_hbm, x0_hbm, ssv, stdv, x0v):
    wid = lax.axis_index("c") * _NS + lax.axis_index("s")
    rows = n_pad // _NW
    base0 = wid * rows

    def chunk(ch, carry):
        base = base0 + ch * _CR
        pltpu.sync_copy(ss_hbm.at[pl.ds(base, _CR)], ssv)
        pltpu.sync_copy(std_hbm.at[pl.ds(base, _CR)], stdv)

        def row(r, c2):
            splat, io = _row_idx(r)
            f = plsc.load_gather(stdv, [splat])
            x = plsc.load_gather(ssv, [splat, io])
            plsc.store_scatter(x0v, [splat, io], f * x)
            return c2

        lax.fori_loop(0, _CR, row, 0, unroll=4)
        pltpu.sync_copy(x0v, x0_hbm.at[pl.ds(base, _CR)])
        return carry

    lax.fori_loop(0, rows // _CR, chunk, 0)


def _sc_prep(ss2, std, n_pad):
    mesh = plsc.VectorSubcoreMesh(core_axis_name="c", subcore_axis_name="s")
    out_type = jax.ShapeDtypeStruct((n_pad, _LANES), jnp.float32)
    scratch = [
        pltpu.VMEM((_CR, _LANES), jnp.float32),
        pltpu.VMEM((_CR,), jnp.float32),
        pltpu.VMEM((_CR, _LANES), jnp.float32),
    ]
    body = functools.partial(_sc_prep_body, n_pad)
    return pl.kernel(
        body, out_type=out_type, mesh=mesh, scratch_types=scratch,
        compiler_params=pltpu.CompilerParams(use_tc_tiling_on_sc=False,
                                            needs_layout_passes=False),
    )(ss2, std)


def _sc_combine_body(n_pad, x_hbm, raw_hbm, a_hbm, b_hbm, xn_hbm,
                     xv, r0v, r1v, av, bv, xnv, sin0, sin1, sout):
    wid = lax.axis_index("c") * _NS + lax.axis_index("s")
    rows = n_pad // _NW
    base0 = wid * rows
    nch = rows // _CR
    sin = (sin0, sin1)

    pltpu.sync_copy(a_hbm.at[pl.ds(base0, rows)], av)
    pltpu.sync_copy(b_hbm.at[pl.ds(base0, rows)], bv)

    def fire_in(b, ch):
        base = base0 + ch * _CR
        pltpu.async_copy(x_hbm.at[pl.ds(base, _CR)], xv.at[b], sin[b])
        pltpu.async_copy(raw_hbm.at[0, pl.ds(base, _CR)], r0v.at[b], sin[b])
        pltpu.async_copy(raw_hbm.at[1, pl.ds(base, _CR)], r1v.at[b], sin[b])

    def wait_in(b):
        for dst in (xv.at[b], r0v.at[b], r1v.at[b]):
            pltpu.make_async_copy(x_hbm.at[pl.ds(0, _CR)], dst, sin[b]).wait()

    def drain_out(b):
        pltpu.make_async_copy(x_hbm.at[pl.ds(0, _CR)], xnv.at[b],
                              sout).wait()

    fire_in(0, 0)
    for ch in range(nch):
        b = ch % 2
        if ch + 1 < nch:
            fire_in(1 - b, ch + 1)
        wait_in(b)
        if ch >= 2:
            drain_out(b)
        abase = ch * _CR

        def row(r, c2):
            splat, io = _row_idx(r)
            asp = jnp.full((16,), abase, jnp.int32) + splat
            bsp = jnp.full((16,), b, jnp.int32)
            fa = plsc.load_gather(av, [asp])
            fb = plsc.load_gather(bv, [asp])
            x = plsc.load_gather(xv, [bsp, splat, io])
            r0 = plsc.load_gather(r0v, [bsp, splat, io])
            r1 = plsc.load_gather(r1v, [bsp, splat, io])
            plsc.store_scatter(xnv, [bsp, splat, io],
                               fa * x + fb * (r0 + r1))
            return c2

        lax.fori_loop(0, _CR, row, 0, unroll=4)
        pltpu.async_copy(xnv.at[b], xn_hbm.at[pl.ds(base0 + ch * _CR, _CR)],
                         sout)
    for _ in range(min(nch, 2)):
        drain_out(0)


def _sc_combine(x, raw, a, b, n_pad):
    mesh = plsc.VectorSubcoreMesh(core_axis_name="c", subcore_axis_name="s")
    rows = n_pad // _NW
    out_type = jax.ShapeDtypeStruct((n_pad, _LANES), jnp.float32)
    scratch = (
        [pltpu.VMEM((2, _CR, _LANES), jnp.float32)] * 3
        + [pltpu.VMEM((rows,), jnp.float32)] * 2
        + [pltpu.VMEM((2, _CR, _LANES), jnp.float32)]
        + [pltpu.SemaphoreType.DMA] * 3
    )
    body = functools.partial(_sc_combine_body, n_pad)
    return pl.kernel(
        body, out_type=out_type, mesh=mesh, scratch_types=scratch,
        compiler_params=pltpu.CompilerParams(use_tc_tiling_on_sc=False,
                                            needs_layout_passes=False),
    )(x, raw, a, b)


def _sc_final_body(n_pad, x_hbm, raw_hbm, a_hbm, b_hbm, p_hbm, m_hbm, o_hbm,
                   xv, r0v, r1v, av, bv, pv, mv, ov, sin0, sin1, sout):
    wid = lax.axis_index("c") * _NS + lax.axis_index("s")
    rows = n_pad // _NW
    base0 = wid * rows
    nch = rows // _CR
    sin = (sin0, sin1)

    pltpu.sync_copy(a_hbm.at[pl.ds(base0, rows)], av)
    pltpu.sync_copy(b_hbm.at[pl.ds(base0, rows)], bv)
    pltpu.sync_copy(p_hbm.at[pl.ds(base0, rows)], pv)
    pltpu.sync_copy(m_hbm.at[pl.ds(base0, rows)], mv)

    def fire_in(b, ch):
        base = base0 + ch * _CR
        pltpu.async_copy(x_hbm.at[pl.ds(base, _CR)], xv.at[b], sin[b])
        pltpu.async_copy(raw_hbm.at[0, pl.ds(base, _CR)], r0v.at[b], sin[b])
        pltpu.async_copy(raw_hbm.at[1, pl.ds(base, _CR)], r1v.at[b], sin[b])

    def wait_in(b):
        for dst in (xv.at[b], r0v.at[b], r1v.at[b]):
            pltpu.make_async_copy(x_hbm.at[pl.ds(0, _CR)], dst, sin[b]).wait()

    def drain_out(b):
        pltpu.make_async_copy(o_hbm.at[:, pl.ds(0, _CR)],
                              ov.at[b], sout).wait()

    fire_in(0, 0)
    for ch in range(nch):
        b = ch % 2
        if ch + 1 < nch:
            fire_in(1 - b, ch + 1)
        wait_in(b)
        if ch >= 2:
            drain_out(b)
        abase = ch * _CR

        def row(r, c2):
            splat, io = _row_idx(r)
            asp = jnp.full((16,), abase, jnp.int32) + splat
            bsp = jnp.full((16,), b, jnp.int32)
            fa = plsc.load_gather(av, [asp])
            fb = plsc.load_gather(bv, [asp])
            fp = plsc.load_gather(pv, [asp])
            fm = plsc.load_gather(mv, [asp])
            x = plsc.load_gather(xv, [bsp, splat, io])
            r0 = plsc.load_gather(r0v, [bsp, splat, io])
            r1 = plsc.load_gather(r1v, [bsp, splat, io])
            x2 = fa * x + fb * (r0 + r1)
            plsc.store_scatter(ov, [bsp, io, splat], fp * x2 + fm)
            return c2

        lax.fori_loop(0, _CR, row, 0, unroll=4)
        pltpu.async_copy(ov.at[b],
                         o_hbm.at[:, pl.ds(base0 + ch * _CR, _CR)], sout)
    for _ in range(min(nch, 2)):
        drain_out(0)


def _sc_final(x, raw, a, b, p, m, n_pad):
    mesh = plsc.VectorSubcoreMesh(core_axis_name="c", subcore_axis_name="s")
    rows = n_pad // _NW
    out_type = jax.ShapeDtypeStruct((_LANES, n_pad), jnp.float32)
    scratch = (
        [pltpu.VMEM((2, _CR, _LANES), jnp.float32)] * 3
        + [pltpu.VMEM((rows,), jnp.float32)] * 4
        + [pltpu.VMEM((2, _LANES, _CR), jnp.float32)]
        + [pltpu.SemaphoreType.DMA] * 3
    )
    body = functools.partial(_sc_final_body, n_pad)
    return pl.kernel(
        body, out_type=out_type, mesh=mesh, scratch_types=scratch,
        compiler_params=pltpu.CompilerParams(use_tc_tiling_on_sc=False,
                                            needs_layout_passes=False),
    )(x, raw, a, b, p, m)


def _factor_body(params_ref, deg0_ref, deg1_ref, diag_ref, pdiag_ref,
                 std_ref, a1_ref, b1_ref, a2_ref, b2_ref, pdq_ref):
    s1, n1, g1 = params_ref[0], params_ref[1], params_ref[2]
    s2, n2, g2 = params_ref[3], params_ref[4], params_ref[5]
    deg = jnp.maximum(deg0_ref[...] + deg1_ref[...], 1.0)
    ld = jnp.log(deg)
    a1_ref[...] = s1 * jnp.exp(g1 * ld)
    b1_ref[...] = n1 * jnp.exp((g1 - 1.0) * ld)
    a2_ref[...] = s2 * jnp.exp(g2 * ld)
    b2_ref[...] = n2 * jnp.exp((g2 - 1.0) * ld)
    std_ref[...] = jax.nn.softplus(diag_ref[...])
    pdq_ref[...] = jax.nn.softplus(pdiag_ref[...])


def _std_body(diag_ref, std_ref):
    std_ref[...] = jax.nn.softplus(diag_ref[...])


def _tc_std(diag_pad, n_pad):
    m = n_pad // 128
    full = pl.BlockSpec((m, 128), lambda: (0, 0))
    out = pl.pallas_call(
        _std_body, in_specs=[full], out_specs=full,
        out_shape=jax.ShapeDtypeStruct((m, 128), jnp.float32),
    )(diag_pad.reshape(m, 128))
    return out.reshape(n_pad)


def _tc_factors(params, deg0, deg1, diag_pad, pdiag_pad, n_pad):
    m = n_pad // 128
    shp = jax.ShapeDtypeStruct((m, 128), jnp.float32)
    full = pl.BlockSpec((m, 128), lambda: (0, 0))
    outs = pl.pallas_call(
        _factor_body,
        in_specs=[pl.BlockSpec(memory_space=pltpu.SMEM)] + [full] * 4,
        out_specs=[full] * 6,
        out_shape=[shp] * 6,
    )(params, deg0.reshape(m, 128), deg1.reshape(m, 128),
      diag_pad.reshape(m, 128), pdiag_pad.reshape(m, 128))
    return [o.reshape(n_pad) for o in outs]


def kernel(standard_sample, mean_param, diag_param, post_diag_param,
           alpha1, alpha2, gamma_param, edge_index):
    S, N = standard_sample.shape
    E = edge_index.shape[1]
    bn = 2048
    n_pad = ((N + 1 + bn - 1) // bn) * bn
    e_align = _NW * _CHUNK * _WIN * 2
    e_pad = ((E + e_align - 1) // e_align) * e_align

    ss_t = jnp.pad(standard_sample.T, ((0, n_pad - N), (0, _LANES - S)))
    diag_pad = jnp.pad(diag_param, (0, n_pad - N))
    pdiag_pad = jnp.pad(post_diag_param, (0, n_pad - N))
    mean_pad = jnp.pad(mean_param, (0, n_pad - N))
    src2 = jnp.pad(edge_index[0], (0, e_pad - E),
                   constant_values=N).reshape(-1, _CHUNK)
    dst2 = jnp.pad(edge_index[1], (0, e_pad - E),
                   constant_values=N).reshape(-1, _CHUNK)
    sw = jnp.exp(alpha1)
    nw = sw * jnp.tanh(alpha2)
    g = jax.nn.sigmoid(gamma_param)
    params = jnp.stack([sw[0], nw[0], g[0], sw[1], nw[1], g[1]])

    stdf = _tc_std(diag_pad, n_pad)
    x0 = _sc_prep(ss_t, stdf, n_pad)
    raw1, deg0, deg1 = _sc_edge_pass(x0, src2, dst2, n_pad, e_pad,
                                     with_deg=True)
    _, a1, b1, a2, b2, pdq = _tc_factors(
        params, deg0, deg1, diag_pad, pdiag_pad, n_pad)
    x1 = _sc_combine(x0, raw1, a1, b1, n_pad)
    raw2 = _sc_edge_pass(x1, src2, dst2, n_pad, e_pad)
    out_t = _sc_final(x1, raw2, a2, b2, pdq, mean_pad, n_pad)
    return out_t[:S, :N]
